# baseline (speedup 1.0000x reference)
import jax
import jax.numpy as jnp
from jax import lax
from jax.experimental import pallas as pl
from jax.experimental.pallas import tpu as pltpu
from jax.experimental.pallas import tpu_sc as plsc

_NUM_SEG = 10000
_N_ROWS = 320000
_D = 128
_NC = 2
_NS = 16
_HALF = _D // _NC
_CHUNK = 128
_NBUF = 2
_MAIN_CHUNKS = 156
_ROWS_PER_TILE = _MAIN_CHUNKS * _CHUNK
_TAIL_BASE = _ROWS_PER_TILE * _NS
_TAIL_CHUNKS = (_N_ROWS - _TAIL_BASE) // _CHUNK
_SEG_PER_TILE = _NUM_SEG // _NS
_ZSTEP = _SEG_PER_TILE // 5


def _sc_body(data_hbm, seg_hbm, sums_out, sqs_out, cnt_out,
             ids_v, comb_v, ones_v,
             comb_acc, cnt_acc,
             ld0, ld1):
    c = lax.axis_index("c")
    s = lax.axis_index("s")
    col0 = c * _HALF
    ld_sems = [ld0, ld1]

    zeros16 = jnp.zeros((16,), jnp.float32)
    ones16 = jnp.ones((16,), jnp.float32)

    def _zfill(i, carry):
        for j in range(2 * _HALF // 16):
            comb_v[0, i, pl.ds(j * 16, 16)] = zeros16
        return carry

    lax.fori_loop(0, _ZSTEP, _zfill, 0)

    def _ofill(i, carry):
        ones_v[i, :] = ones16
        return carry

    lax.fori_loop(0, _CHUNK, _ofill, 0)

    r0 = s * _SEG_PER_TILE
    zrow = comb_v.at[0, pl.ds(0, _ZSTEP)]
    zcnt = comb_v.at[0, pl.ds(0, _ZSTEP), pl.ds(0, 16)]
    for k in range(5):
        sl = pl.ds(r0 + k * _ZSTEP, _ZSTEP)
        pltpu.sync_copy(zrow, comb_acc.at[sl])
        pltpu.sync_copy(zcnt, cnt_acc.at[sl])

    plsc.subcore_barrier()

    def _issue(chunk_idx, b):
        row0 = s * _ROWS_PER_TILE + chunk_idx * _CHUNK
        pltpu.async_copy(seg_hbm.at[pl.ds(row0, _CHUNK)], ids_v.at[b],
                         ld_sems[b])
        pltpu.async_copy(
            data_hbm.at[pl.ds(row0, _CHUNK), pl.ds(col0, _HALF)],
            comb_v.at[b, :, pl.ds(0, _HALF)], ld_sems[b])

    def _wait_load(b):
        pltpu.make_async_copy(
            seg_hbm.at[pl.ds(0, _CHUNK)], ids_v.at[b], ld_sems[b]).wait()
        pltpu.make_async_copy(
            data_hbm.at[pl.ds(0, _CHUNK), pl.ds(0, _HALF)],
            comb_v.at[b, :, pl.ds(0, _HALF)], ld_sems[b]).wait()

    def _square(b):
        def _sq(i, cc):
            for jj in range(_HALF // 16):
                v = comb_v[b, i, pl.ds(jj * 16, 16)]
                comb_v[b, i, pl.ds(_HALF + jj * 16, 16)] = v * v
            return cc

        lax.fori_loop(0, _CHUNK, _sq, 0)


    _issue(0, 0)

    def _outer(jo, carry):
        for b in range(_NBUF):
            j = jo * _NBUF + b
            pf = j + 1
            bpf = b ^ 1

            @pl.when(pf < _MAIN_CHUNKS)
            def _():
                _issue(pf, bpf)

            _wait_load(b)
            _square(b)
            pltpu.sync_copy(comb_v.at[b], comb_acc.at[ids_v.at[b]], add=True)

            @pl.when(c == 0)
            def _():
                pltpu.sync_copy(ones_v, cnt_acc.at[ids_v.at[b]], add=True)
        return carry

    lax.fori_loop(0, _MAIN_CHUNKS // _NBUF, _outer, 0)

    @pl.when(s < _TAIL_CHUNKS)
    def _():
        row0 = _TAIL_BASE + s * _CHUNK
        pltpu.sync_copy(seg_hbm.at[pl.ds(row0, _CHUNK)], ids_v.at[0])
        pltpu.sync_copy(
            data_hbm.at[pl.ds(row0, _CHUNK), pl.ds(col0, _HALF)],
            comb_v.at[0, :, pl.ds(0, _HALF)])
        _square(0)
        pltpu.sync_copy(comb_v.at[0], comb_acc.at[ids_v.at[0]], add=True)

    @pl.when(jnp.logical_and(c == 0, s < _TAIL_CHUNKS))
    def _():
        pltpu.sync_copy(ones_v, cnt_acc.at[ids_v.at[0]], add=True)

    plsc.subcore_barrier()

    out_sl = pl.ds(r0, _SEG_PER_TILE)
    pltpu.sync_copy(comb_acc.at[out_sl, pl.ds(0, _HALF)],
                    sums_out.at[out_sl, pl.ds(col0, _HALF)])
    pltpu.sync_copy(comb_acc.at[out_sl, pl.ds(_HALF, _HALF)],
                    sqs_out.at[out_sl, pl.ds(col0, _HALF)])

    @pl.when(c == 0)
    def _():
        pltpu.sync_copy(cnt_acc.at[out_sl], cnt_out.at[out_sl])


_sc_accumulate = pl.kernel(
    _sc_body,
    out_type=(
        jax.ShapeDtypeStruct((_NUM_SEG, _D), jnp.float32),
        jax.ShapeDtypeStruct((_NUM_SEG, _D), jnp.float32),
        jax.ShapeDtypeStruct((_NUM_SEG, 16), jnp.float32),
    ),
    mesh=plsc.VectorSubcoreMesh(core_axis_name="c", subcore_axis_name="s"),
    compiler_params=pltpu.CompilerParams(use_tc_tiling_on_sc=False),
    scratch_types=[
        pltpu.VMEM((_NBUF, _CHUNK), jnp.int32),
        pltpu.VMEM((_NBUF, _CHUNK, 2 * _HALF), jnp.float32),
        pltpu.VMEM((_CHUNK, 16), jnp.float32),
        pltpu.VMEM_SHARED((_NUM_SEG, 2 * _HALF), jnp.float32),
        pltpu.VMEM_SHARED((_NUM_SEG, 16), jnp.float32),
        pltpu.SemaphoreType.DMA,
        pltpu.SemaphoreType.DMA,
    ],
)


def _fin_body(sums_ref, sqs_ref, cnt_ref, out_ref):
    cnt = cnt_ref[:, 0:1]
    safe = jnp.maximum(cnt, 1.0)
    mean = sums_ref[...] / safe
    var = (sqs_ref[...] - cnt * mean * mean) / jnp.maximum(cnt - 1.0, 1.0)
    var = jnp.maximum(var, 0.0)
    sem = jnp.sqrt(var / safe + 1e-12)
    out_ref[:, : _D] = mean
    out_ref[:, _D:] = sem


_FIN_ROWS = 1000

_finalize = pl.pallas_call(
    _fin_body,
    grid=(_NUM_SEG // _FIN_ROWS,),
    in_specs=[
        pl.BlockSpec((_FIN_ROWS, _D), lambda i: (i, 0)),
        pl.BlockSpec((_FIN_ROWS, _D), lambda i: (i, 0)),
        pl.BlockSpec((_FIN_ROWS, 16), lambda i: (i, 0)),
    ],
    out_specs=pl.BlockSpec((_FIN_ROWS, 2 * _D), lambda i: (i, 0)),
    out_shape=jax.ShapeDtypeStruct((_NUM_SEG, 2 * _D), jnp.float32),
)


def kernel(data, segment_ids):
    seg = segment_ids.astype(jnp.int32)
    sums, sqs, cnt = _sc_accumulate(data, seg)
    return _finalize(sums, sqs, cnt)


# speedup vs baseline: 1.4189x; 1.4189x over previous
import jax
import jax.numpy as jnp
from jax import lax
from jax.experimental import pallas as pl
from jax.experimental.pallas import tpu as pltpu
from jax.experimental.pallas import tpu_sc as plsc

_NUM_SEG = 10000
_N_ROWS = 320000
_D = 128
_NC = 2
_NS = 16
_HALF = _D // _NC
_CHUNK = 128
_NBUF = 2
_MAIN_CHUNKS = 156
_ROWS_PER_TILE = _MAIN_CHUNKS * _CHUNK
_TAIL_BASE = _ROWS_PER_TILE * _NS
_TAIL_CHUNKS = (_N_ROWS - _TAIL_BASE) // _CHUNK
_SEG_PER_TILE = _NUM_SEG // _NS
_ZSTEP = _SEG_PER_TILE // 5


def _sc_body(data_hbm, seg_hbm, sums_out, sqs_out, cnt_out,
             ids_v, rows_v, sq_v, ones_v,
             sums_acc, sqs_acc, cnt_acc,
             ld0, ld1, sc0, sc1):
    c = lax.axis_index("c")
    s = lax.axis_index("s")
    col0 = c * _HALF
    ld_sems = [ld0, ld1]
    sc_sems = [sc0, sc1]

    zeros16 = jnp.zeros((16,), jnp.float32)
    ones16 = jnp.ones((16,), jnp.float32)

    def _zfill(i, carry):
        for j in range(_HALF // 16):
            sq_v[0, i, pl.ds(j * 16, 16)] = zeros16
        return carry

    lax.fori_loop(0, _ZSTEP, _zfill, 0)

    def _ofill(i, carry):
        ones_v[i, :] = ones16
        return carry

    lax.fori_loop(0, _CHUNK, _ofill, 0)

    r0 = s * _SEG_PER_TILE
    zrow = sq_v.at[0, pl.ds(0, _ZSTEP)]
    zcnt = sq_v.at[0, pl.ds(0, _ZSTEP), pl.ds(0, 16)]
    for k in range(5):
        sl = pl.ds(r0 + k * _ZSTEP, _ZSTEP)
        pltpu.sync_copy(zrow, sums_acc.at[sl])
        pltpu.sync_copy(zrow, sqs_acc.at[sl])
        pltpu.sync_copy(zcnt, cnt_acc.at[sl])

    plsc.subcore_barrier()

    def _issue(chunk_idx, b):
        row0 = s * _ROWS_PER_TILE + chunk_idx * _CHUNK
        pltpu.async_copy(seg_hbm.at[pl.ds(row0, _CHUNK)], ids_v.at[b],
                         ld_sems[b])
        pltpu.async_copy(
            data_hbm.at[pl.ds(row0, _CHUNK), pl.ds(col0, _HALF)],
            rows_v.at[b], ld_sems[b])

    def _wait_load(b):
        pltpu.make_async_copy(
            seg_hbm.at[pl.ds(0, _CHUNK)], ids_v.at[b], ld_sems[b]).wait()
        pltpu.make_async_copy(
            data_hbm.at[pl.ds(0, _CHUNK), pl.ds(0, _HALF)],
            rows_v.at[b], ld_sems[b]).wait()

    def _square(b):
        def _sq(i, cc):
            for jj in range(_HALF // 16):
                v = rows_v[b, i, pl.ds(jj * 16, 16)]
                sq_v[b, i, pl.ds(jj * 16, 16)] = v * v
            return cc

        lax.fori_loop(0, _CHUNK, _sq, 0)

    def _drain_scatters(b):
        pltpu.make_async_copy(rows_v.at[b], sums_acc.at[ids_v.at[b]],
                              sc_sems[b]).wait()
        pltpu.make_async_copy(sq_v.at[b], sqs_acc.at[ids_v.at[b]],
                              sc_sems[b]).wait()

        @pl.when(c == 0)
        def _():
            pltpu.make_async_copy(ones_v, cnt_acc.at[ids_v.at[b]],
                                  sc_sems[b]).wait()

    _issue(0, 0)

    def _outer(jo, carry):
        for b in range(_NBUF):
            j = jo * _NBUF + b
            pf = j + 1
            bpf = b ^ 1

            @pl.when(pf < _MAIN_CHUNKS)
            def _():
                @pl.when(j >= 1)
                def _():
                    _drain_scatters(bpf)

                _issue(pf, bpf)

            _wait_load(b)
            pltpu.async_copy(rows_v.at[b], sums_acc.at[ids_v.at[b]],
                             sc_sems[b], add=True)

            @pl.when(c == 0)
            def _():
                pltpu.async_copy(ones_v, cnt_acc.at[ids_v.at[b]], sc_sems[b],
                                 add=True)

            _square(b)
            pltpu.async_copy(sq_v.at[b], sqs_acc.at[ids_v.at[b]], sc_sems[b],
                             add=True)
        return carry

    lax.fori_loop(0, _MAIN_CHUNKS // _NBUF, _outer, 0)

    _drain_scatters(0)
    _drain_scatters(1)

    @pl.when(s < _TAIL_CHUNKS)
    def _():
        row0 = _TAIL_BASE + s * _CHUNK
        pltpu.sync_copy(seg_hbm.at[pl.ds(row0, _CHUNK)], ids_v.at[0])
        pltpu.sync_copy(
            data_hbm.at[pl.ds(row0, _CHUNK), pl.ds(col0, _HALF)],
            rows_v.at[0])
        _square(0)
        pltpu.sync_copy(rows_v.at[0], sums_acc.at[ids_v.at[0]], add=True)
        pltpu.sync_copy(sq_v.at[0], sqs_acc.at[ids_v.at[0]], add=True)

    @pl.when(jnp.logical_and(c == 0, s < _TAIL_CHUNKS))
    def _():
        pltpu.sync_copy(ones_v, cnt_acc.at[ids_v.at[0]], add=True)

    plsc.subcore_barrier()

    out_sl = pl.ds(r0, _SEG_PER_TILE)
    pltpu.sync_copy(sums_acc.at[out_sl],
                    sums_out.at[out_sl, pl.ds(col0, _HALF)])
    pltpu.sync_copy(sqs_acc.at[out_sl],
                    sqs_out.at[out_sl, pl.ds(col0, _HALF)])

    @pl.when(c == 0)
    def _():
        pltpu.sync_copy(cnt_acc.at[out_sl], cnt_out.at[out_sl])


_sc_accumulate = pl.kernel(
    _sc_body,
    out_type=(
        jax.ShapeDtypeStruct((_NUM_SEG, _D), jnp.float32),
        jax.ShapeDtypeStruct((_NUM_SEG, _D), jnp.float32),
        jax.ShapeDtypeStruct((_NUM_SEG, 16), jnp.float32),
    ),
    mesh=plsc.VectorSubcoreMesh(core_axis_name="c", subcore_axis_name="s"),
    compiler_params=pltpu.CompilerParams(use_tc_tiling_on_sc=False),
    scratch_types=[
        pltpu.VMEM((_NBUF, _CHUNK), jnp.int32),
        pltpu.VMEM((_NBUF, _CHUNK, _HALF), jnp.float32),
        pltpu.VMEM((_NBUF, _CHUNK, _HALF), jnp.float32),
        pltpu.VMEM((_CHUNK, 16), jnp.float32),
        pltpu.VMEM_SHARED((_NUM_SEG, _HALF), jnp.float32),
        pltpu.VMEM_SHARED((_NUM_SEG, _HALF), jnp.float32),
        pltpu.VMEM_SHARED((_NUM_SEG, 16), jnp.float32),
        pltpu.SemaphoreType.DMA,
        pltpu.SemaphoreType.DMA,
        pltpu.SemaphoreType.DMA,
        pltpu.SemaphoreType.DMA,
    ],
)


def _fin_body(sums_ref, sqs_ref, cnt_ref, out_ref):
    cnt = cnt_ref[:, 0:1]
    safe = jnp.maximum(cnt, 1.0)
    mean = sums_ref[...] / safe
    var = (sqs_ref[...] - cnt * mean * mean) / jnp.maximum(cnt - 1.0, 1.0)
    var = jnp.maximum(var, 0.0)
    sem = jnp.sqrt(var / safe + 1e-12)
    out_ref[:, : _D] = mean
    out_ref[:, _D:] = sem


_FIN_ROWS = 1000

_finalize = pl.pallas_call(
    _fin_body,
    grid=(_NUM_SEG // _FIN_ROWS,),
    in_specs=[
        pl.BlockSpec((_FIN_ROWS, _D), lambda i: (i, 0)),
        pl.BlockSpec((_FIN_ROWS, _D), lambda i: (i, 0)),
        pl.BlockSpec((_FIN_ROWS, 16), lambda i: (i, 0)),
    ],
    out_specs=pl.BlockSpec((_FIN_ROWS, 2 * _D), lambda i: (i, 0)),
    out_shape=jax.ShapeDtypeStruct((_NUM_SEG, 2 * _D), jnp.float32),
)


def kernel(data, segment_ids):
    seg = segment_ids.astype(jnp.int32)
    sums, sqs, cnt = _sc_accumulate(data, seg)
    return _finalize(sums, sqs, cnt)
